# Initial kernel scaffold; baseline (speedup 1.0000x reference)
import jax
import jax.numpy as jnp
from jax.experimental import pallas as pl


def kernel(x, table, W, b):
    raise NotImplementedError("write your pallas kernel here")


# speedup vs baseline: 67.4437x; 67.4437x over previous
import functools

import jax
import jax.numpy as jnp
from jax import lax
from jax.experimental import pallas as pl
from jax.experimental.pallas import tpu as pltpu
from jax.experimental.pallas import tpu_sc as plsc

VOCAB = 256
EMBED_DIM = 32
NUM_CLASSES = 100
BATCH = 16384
SEQ = 200

NC, NS, L = 2, 16, 16
NW = NC * NS
ROWS_PER_W = BATCH // NW
GROUPS = ROWS_PER_W // L
XCHUNK = L * SEQ
HCHUNK = L * VOCAB


def _sc_histogram(x_flat):
    mesh = plsc.VectorSubcoreMesh(
        core_axis_name="c", subcore_axis_name="s",
        num_cores=NC, num_subcores=NS)

    @functools.partial(
        pl.kernel,
        out_type=jax.ShapeDtypeStruct((BATCH * VOCAB,), jnp.float32),
        mesh=mesh,
        compiler_params=pltpu.CompilerParams(needs_layout_passes=False),
        scratch_types=[
            pltpu.VMEM((XCHUNK,), jnp.int32),
            pltpu.VMEM((HCHUNK,), jnp.float32),
        ],
    )
    def hist_kernel(x_hbm, out_hbm, x_v, hist_v):
        wid = lax.axis_index("s") * NC + lax.axis_index("c")
        lane = lax.iota(jnp.int32, L)
        base_x = lane * SEQ
        base_h = lane * VOCAB
        ones = jnp.full((L,), 1.0, jnp.float32)
        zeros = jnp.zeros((L,), jnp.float32)

        def group_body(g, carry):
            gid = wid * GROUPS + g
            pltpu.sync_copy(x_hbm.at[pl.ds(gid * XCHUNK, XCHUNK)], x_v)
            for i in range(HCHUNK // L):
                hist_v[pl.ds(i * L, L)] = zeros
            for s in range(SEQ):
                vals = plsc.load_gather(x_v, [base_x + s])
                plsc.addupdate_scatter(hist_v, [base_h + vals], ones)
            pltpu.sync_copy(hist_v, out_hbm.at[pl.ds(gid * HCHUNK, HCHUNK)])
            return carry

        lax.fori_loop(0, GROUPS, group_body, 0)

    return hist_kernel(x_flat)


def _tc_classify(counts, table, W, b):
    BLK = 512

    def body(c_ref, t_ref, w_ref, b_ref, o_ref):
        m = jnp.dot(t_ref[...], w_ref[...],
                    preferred_element_type=jnp.float32)
        out = jnp.dot(c_ref[...], m,
                      preferred_element_type=jnp.float32)
        o_ref[...] = out * (1.0 / SEQ) + b_ref[...]

    return pl.pallas_call(
        body,
        grid=(BATCH // BLK,),
        in_specs=[
            pl.BlockSpec((BLK, VOCAB), lambda j: (j, 0)),
            pl.BlockSpec((VOCAB, EMBED_DIM), lambda j: (0, 0)),
            pl.BlockSpec((EMBED_DIM, NUM_CLASSES), lambda j: (0, 0)),
            pl.BlockSpec((1, NUM_CLASSES), lambda j: (0, 0)),
        ],
        out_specs=pl.BlockSpec((BLK, NUM_CLASSES), lambda j: (j, 0)),
        out_shape=jax.ShapeDtypeStruct((BATCH, NUM_CLASSES), jnp.float32),
    )(counts, table, W, b.reshape(1, NUM_CLASSES))


def kernel(x, table, W, b):
    x_flat = x.astype(jnp.int32).reshape(BATCH * SEQ)
    counts = _sc_histogram(x_flat).reshape(BATCH, VOCAB)
    return _tc_classify(counts, table, W, b)
